# baseline (device time: 212763 ns/iter reference)
import jax
import jax.numpy as jnp
from jax import lax
from jax.experimental import pallas as pl
from jax.experimental.pallas import tpu as pltpu

N_DEV = 8
N_CHUNK = 32


def kernel(x):
    m_per, n_cols = x.shape
    n_out = n_cols // N_DEV
    m_out = m_per * N_DEV

    def body(x_ref, out_ref, local_sem, send_sems, recv_sems):
        my = lax.axis_index("i")

        barrier_sem = pltpu.get_barrier_semaphore()
        for h in range(1, N_DEV):
            pl.semaphore_signal(
                barrier_sem, inc=1,
                device_id=((my + h) % N_DEV,),
                device_id_type=pl.DeviceIdType.MESH,
            )
        pl.semaphore_wait(barrier_sem, N_DEV - 1)

        local_copy = pltpu.make_async_copy(
            x_ref.at[:, pl.ds(my * n_out, n_out)],
            out_ref.at[pl.ds(my * m_per, m_per), :],
            local_sem,
        )
        local_copy.start()

        rows = m_per // N_CHUNK
        rdmas = []
        for c in range(N_CHUNK):
            for idx in range(N_DEV - 1):
                h = 1 + (idx + c) % (N_DEV - 1)
                dst = (my + h) % N_DEV
                rdma = pltpu.make_async_remote_copy(
                    src_ref=x_ref.at[
                        pl.ds(c * rows, rows), pl.ds(dst * n_out, n_out)
                    ],
                    dst_ref=out_ref.at[
                        pl.ds(my * m_per + c * rows, rows), :
                    ],
                    send_sem=send_sems.at[h],
                    recv_sem=recv_sems.at[h],
                    device_id=(dst,),
                    device_id_type=pl.DeviceIdType.MESH,
                )
                rdma.start()
                rdmas.append(rdma)

        local_copy.wait()
        for rdma in rdmas:
            rdma.wait()

    return pl.pallas_call(
        body,
        out_shape=jax.ShapeDtypeStruct((m_out, n_out), x.dtype),
        in_specs=[pl.BlockSpec(memory_space=pl.ANY)],
        out_specs=pl.BlockSpec(memory_space=pl.ANY),
        scratch_shapes=[
            pltpu.SemaphoreType.DMA,
            pltpu.SemaphoreType.DMA((N_DEV,)),
            pltpu.SemaphoreType.DMA((N_DEV,)),
        ],
        compiler_params=pltpu.CompilerParams(collective_id=0),
    )(x)


# device time: 210310 ns/iter; 1.0117x vs baseline; 1.0117x over previous
import jax
import jax.numpy as jnp
from jax import lax
from jax.experimental import pallas as pl
from jax.experimental.pallas import tpu as pltpu

N_DEV = 8
N_CHUNK = 32


def kernel(x):
    m_per, n_cols = x.shape
    n_out = n_cols // N_DEV
    m_out = m_per * N_DEV

    def body(x_ref, out_ref, local_sem, send_sems, recv_sems):
        my = lax.axis_index("i")

        barrier_sem = pltpu.get_barrier_semaphore()
        for h in range(1, N_DEV):
            pl.semaphore_signal(
                barrier_sem, inc=1,
                device_id=((my + h) % N_DEV,),
                device_id_type=pl.DeviceIdType.MESH,
            )
        pl.semaphore_wait(barrier_sem, N_DEV - 1)

        local_copy = pltpu.make_async_copy(
            x_ref.at[:, pl.ds(my * n_out, n_out)],
            out_ref.at[pl.ds(my * m_per, m_per), :],
            local_sem,
        )
        local_copy.start()

        rows = m_per // N_CHUNK
        rdmas = []
        for c in range(N_CHUNK):
            for h in range(1, N_DEV):
                dst = (my + h) % N_DEV
                rdma = pltpu.make_async_remote_copy(
                    src_ref=x_ref.at[
                        pl.ds(c * rows, rows), pl.ds(dst * n_out, n_out)
                    ],
                    dst_ref=out_ref.at[
                        pl.ds(my * m_per + c * rows, rows), :
                    ],
                    send_sem=send_sems.at[h],
                    recv_sem=recv_sems.at[h],
                    device_id=(dst,),
                    device_id_type=pl.DeviceIdType.MESH,
                )
                rdma.start()
                rdmas.append(rdma)

        local_copy.wait()
        for rdma in rdmas:
            rdma.wait()

    return pl.pallas_call(
        body,
        out_shape=jax.ShapeDtypeStruct((m_out, n_out), x.dtype),
        in_specs=[pl.BlockSpec(memory_space=pl.ANY)],
        out_specs=pl.BlockSpec(memory_space=pl.ANY),
        scratch_shapes=[
            pltpu.SemaphoreType.DMA,
            pltpu.SemaphoreType.DMA((N_DEV,)),
            pltpu.SemaphoreType.DMA((N_DEV,)),
        ],
        compiler_params=pltpu.CompilerParams(collective_id=0),
    )(x)


# device time: 209622 ns/iter; 1.0150x vs baseline; 1.0033x over previous
import jax
import jax.numpy as jnp
from jax import lax
from jax.experimental import pallas as pl
from jax.experimental.pallas import tpu as pltpu

N_DEV = 8
N_CHUNK = 32


def kernel(x):
    m_per, n_cols = x.shape
    n_out = n_cols // N_DEV
    m_out = m_per * N_DEV

    def body(x_ref, out_ref, local_sem, send_sems, recv_sems):
        my = lax.axis_index("i")

        barrier_sem = pltpu.get_barrier_semaphore()
        for h in range(1, N_DEV):
            pl.semaphore_signal(
                barrier_sem, inc=1,
                device_id=((my + h) % N_DEV,),
                device_id_type=pl.DeviceIdType.MESH,
            )
        pl.semaphore_wait(barrier_sem, N_DEV - 1)

        local_copy = pltpu.make_async_copy(
            x_ref.at[:, pl.ds(my * n_out, n_out)],
            out_ref.at[pl.ds(my * m_per, m_per), :],
            local_sem,
        )
        local_copy.start()

        rows = m_per // N_CHUNK
        rdmas = []
        h_order = (2, 6, 3, 5, 1, 7, 4)
        for c in range(N_CHUNK):
            for h in h_order:
                dst = (my + h) % N_DEV
                rdma = pltpu.make_async_remote_copy(
                    src_ref=x_ref.at[
                        pl.ds(c * rows, rows), pl.ds(dst * n_out, n_out)
                    ],
                    dst_ref=out_ref.at[
                        pl.ds(my * m_per + c * rows, rows), :
                    ],
                    send_sem=send_sems.at[h],
                    recv_sem=recv_sems.at[h],
                    device_id=(dst,),
                    device_id_type=pl.DeviceIdType.MESH,
                )
                rdma.start()
                rdmas.append(rdma)

        local_copy.wait()
        for rdma in rdmas:
            rdma.wait()

    return pl.pallas_call(
        body,
        out_shape=jax.ShapeDtypeStruct((m_out, n_out), x.dtype),
        in_specs=[pl.BlockSpec(memory_space=pl.ANY)],
        out_specs=pl.BlockSpec(memory_space=pl.ANY),
        scratch_shapes=[
            pltpu.SemaphoreType.DMA,
            pltpu.SemaphoreType.DMA((N_DEV,)),
            pltpu.SemaphoreType.DMA((N_DEV,)),
        ],
        compiler_params=pltpu.CompilerParams(collective_id=0),
    )(x)
